# Initial kernel scaffold; baseline (speedup 1.0000x reference)
import jax
import jax.numpy as jnp
from jax.experimental import pallas as pl


def kernel(x0, x1, x2, x3, x4, x5, x6, x7):
    raise NotImplementedError("write your pallas kernel here")


# speedup vs baseline: 1.7696x; 1.7696x over previous
import functools

import jax
import jax.numpy as jnp
from jax import lax
from jax.experimental import pallas as pl
from jax.experimental.pallas import tpu as pltpu
from jax.experimental.pallas import tpu_sc as plsc

N = 1048576
NP = 8
K = N // 2 + 1
NW = 16
ROWS_W = N // NW
CHUNK = 2048
NCH = ROWS_W // CHUNK
CAP = 4608
NBIN = 4096
TOTAL_AMOUNT = 30000000.0
MIN_AMOUNT = 1500.0
QUORUM = 17.0

_I16 = lambda v: jnp.full((16,), v, jnp.int32)


def _select_kth(hist, stage_i, r0, iota):
    def body(j, carry):
        bstar, kprime, tot = carry
        jj = 255 - j
        v = hist[pl.ds(jj * 16, 16)]
        sv = jnp.sum(v)
        found = (tot < r0) & (r0 <= tot + sv)
        w = plsc.cumsum(lax.rev(v, (0,)))
        r = r0 - tot
        istar = plsc.all_reduce_ffs(w >= r)
        bb = _I16(jj * 16 + 15) - istar
        wst = jnp.take(w, istar)
        vst = jnp.take(v, _I16(15) - istar)
        kp = _I16(1) * (r0 - tot) - wst + vst
        bstar = jnp.where(found, bb, bstar)
        kprime = jnp.where(found, kp, kprime)
        return bstar, kprime, tot + sv
    bstar, kprime, _ = lax.fori_loop(
        0, 256, body, (_I16(0), _I16(1), jnp.int32(0)))
    stage_i[...] = jnp.where(iota == 0, bstar,
                             jnp.where(iota == 1, kprime, _I16(0)))


def _sc_body(x0, x1, x2, x3, x4, x5, x6, x7,
             stats, buckets, bcounts, hists, redh, bsel, parts,
             databuf, lbuckets, lcounts, hist, flags, segbuf,
             redbuf, red256, pbuf, stage_i, stage_f):
    xrefs = (x0, x1, x2, x3, x4, x5, x6, x7)
    c = lax.axis_index("c")
    s = lax.axis_index("s")
    iota = lax.iota(jnp.int32, 16)
    zeros = jnp.zeros((16,), jnp.int32)
    ones = _I16(1)

    def zflags(i, _):
        flags[pl.ds(i * 16, 16)] = zeros
        return 0
    lax.fori_loop(0, 2048, zflags, 0)

    def zhist(i, _):
        hist[pl.ds(i * 16, 16)] = zeros
        return 0

    def reduce_hists():
        for src in range(16):
            pltpu.sync_copy(hists.at[c, src, pl.ds(s * 256, 256)],
                            redbuf.at[src])
        def rbody(j, _):
            acc = zeros
            for src in range(16):
                acc = acc + redbuf[src, pl.ds(j * 16, 16)]
            red256[pl.ds(j * 16, 16)] = acc
            return 0
        lax.fori_loop(0, 16, rbody, 0)
        pltpu.sync_copy(red256, redh.at[c, pl.ds(s * 256, 256)])

    def project_body(pi, _):
        p = c * 4 + pi

        lcounts[...] = zeros
        lax.fori_loop(0, 256, zhist, 0)

        def a_chunk(ch, _):
            off = (s * ROWS_W + ch * CHUNK) * 2
            for j, xr in enumerate(xrefs):
                @pl.when(p == j)
                def _(xr=xr):
                    pltpu.sync_copy(xr.at[pl.ds(off, CHUNK * 2)], databuf)
            def a_vec(v, _):
                rows = v * 32 + iota * 2
                vot = plsc.load_gather(databuf, [rows])
                amt = plsc.load_gather(databuf, [rows + 1])
                m = (vot * 8388608.0).astype(jnp.int32)
                t = m >> 19
                rc, _last = plsc.scan_count(t)
                base = plsc.load_gather(lcounts, [t])
                pos = base + rc - 1
                plsc.store_scatter(lbuckets, [t, pos], m, mask=pos < CAP)
                plsc.addupdate_scatter(lcounts, [t], ones)
                aidx = (amt * 16777216.0).astype(jnp.int32)
                plsc.addupdate_scatter(hist, [aidx >> 12], ones)
                return 0
            lax.fori_loop(0, CHUNK // 16, a_vec, 0)
            return 0
        lax.fori_loop(0, NCH, a_chunk, 0)

        for t in range(16):
            pltpu.sync_copy(lbuckets.at[t], buckets.at[c, s, t])
        lcv = jnp.minimum(lcounts[...], _I16(CAP))
        stage_i[...] = lcv
        pltpu.sync_copy(stage_i, bcounts.at[c, s])
        pltpu.sync_copy(hist, hists.at[c, s])
        plsc.subcore_barrier()

        reduce_hists()
        plsc.subcore_barrier()

        @pl.when(s == 0)
        def _():
            pltpu.sync_copy(redh.at[c], hist)
            _select_kth(hist, stage_i, jnp.int32(K), iota)
            pltpu.sync_copy(stage_i, bsel.at[c, 0])
        plsc.subcore_barrier()

        pltpu.sync_copy(bsel.at[c, 0], stage_i)
        sel = stage_i[...]
        bstar_v = jnp.take(sel, zeros)
        kprime_v = jnp.take(sel, ones)

        lax.fori_loop(0, 256, zhist, 0)
        def a2_chunk(ch, _):
            off = (s * ROWS_W + ch * CHUNK) * 2
            for j, xr in enumerate(xrefs):
                @pl.when(p == j)
                def _(xr=xr):
                    pltpu.sync_copy(xr.at[pl.ds(off, CHUNK * 2)], databuf)
            def a2_vec(v, _):
                rows = v * 32 + iota * 2
                amt = plsc.load_gather(databuf, [rows + 1])
                aidx = (amt * 16777216.0).astype(jnp.int32)
                inbin = (aidx >> 12) == bstar_v
                plsc.addupdate_scatter(hist, [aidx & 4095], ones, mask=inbin)
                return 0
            lax.fori_loop(0, CHUNK // 16, a2_vec, 0)
            return 0
        lax.fori_loop(0, NCH, a2_chunk, 0)
        pltpu.sync_copy(hist, hists.at[c, s])
        plsc.subcore_barrier()

        reduce_hists()
        plsc.subcore_barrier()

        @pl.when(s == 0)
        def _():
            pltpu.sync_copy(redh.at[c], hist)
            _select_kth(hist, stage_i, jnp.max(kprime_v), iota)
            pltpu.sync_copy(stage_i, bsel.at[c, 1])

        lcounts[...] = zeros
        for src in range(16):
            pltpu.sync_copy(bcounts.at[c, src], stage_i)
            cnt = jnp.max(jnp.take(stage_i[...], _I16(1) * s))
            pltpu.sync_copy(buckets.at[c, src, s], segbuf)
            def ingest(v, _):
                valid = (v * 16 + iota) < cnt
                m = segbuf[pl.ds(v * 16, 16)]
                lid = m & 0x7FFFF
                sb = lid >> 15
                rc, _last = plsc.scan_count(sb, valid)
                base = plsc.load_gather(lcounts, [sb])
                pos = base + rc - 1
                plsc.store_scatter(lbuckets, [sb, pos], lid,
                                   mask=valid & (pos < CAP))
                plsc.addupdate_scatter(lcounts, [sb], ones, mask=valid)
                return 0
            lax.fori_loop(0, (cnt + 15) >> 4, ingest, 0)

        lcv = jnp.minimum(lcounts[...], _I16(CAP))
        acc0 = jnp.int32(0)
        for sb in range(16):
            cnt = jnp.max(jnp.where(iota == sb, lcv, zeros))
            tag = _I16(pi * 16 + sb + 1)
            def fbody(v, acc):
                valid = (v * 16 + iota) < cnt
                lid = lbuckets[sb, pl.ds(v * 16, 16)]
                f = lid & 0x7FFF
                old = plsc.load_gather(flags, [f], mask=valid)
                new = valid & (old != tag)
                _rc, lastm = plsc.scan_count(lid, new)
                inc = plsc.all_reduce_population_count(lastm)
                plsc.store_scatter(flags, [f], tag, mask=new)
                return acc + jnp.max(inc)
            acc0 = lax.fori_loop(0, (cnt + 15) >> 4, fbody, acc0)

        stage_i[...] = _I16(1) * acc0
        pltpu.sync_copy(stage_i, parts.at[c, s])
        plsc.subcore_barrier()

        @pl.when(s == 0)
        def _():
            pltpu.sync_copy(parts.at[c], pbuf)
            total = zeros
            for src in range(16):
                total = total + jnp.where(iota == 0, pbuf[src, :], zeros)
            pltpu.sync_copy(bsel.at[c, 0], stage_i)
            b1 = jnp.take(stage_i[...], zeros)
            pltpu.sync_copy(bsel.at[c, 1], stage_i)
            b2 = jnp.take(stage_i[...], zeros)
            med = (b1 * 4096 + b2).astype(jnp.float32) * 5.9604644775390625e-08
            cntf = total.astype(jnp.float32)
            stage_f[...] = jnp.where(iota == 0, cntf,
                                     jnp.where(iota == 1, med,
                                               jnp.zeros((16,), jnp.float32)))
            pltpu.sync_copy(stage_f, stats.at[p])
        plsc.subcore_barrier()
        return 0

    lax.fori_loop(0, 4, project_body, 0)


def _sc_stats(*xs):
    mesh = plsc.VectorSubcoreMesh(core_axis_name="c", subcore_axis_name="s")
    kern = pl.kernel(
        _sc_body,
        out_type=(
            jax.ShapeDtypeStruct((NP, 16), jnp.float32),
            jax.ShapeDtypeStruct((2, NW, 16, CAP), jnp.int32),
            jax.ShapeDtypeStruct((2, NW, 16), jnp.int32),
            jax.ShapeDtypeStruct((2, NW, NBIN), jnp.int32),
            jax.ShapeDtypeStruct((2, NBIN), jnp.int32),
            jax.ShapeDtypeStruct((2, 2, 16), jnp.int32),
            jax.ShapeDtypeStruct((2, NW, 16), jnp.int32),
        ),
        mesh=mesh,
        compiler_params=pltpu.CompilerParams(needs_layout_passes=False),
        scratch_types=[
            pltpu.VMEM((CHUNK * 2,), jnp.float32),
            pltpu.VMEM((16, CAP), jnp.int32),
            pltpu.VMEM((16,), jnp.int32),
            pltpu.VMEM((NBIN,), jnp.int32),
            pltpu.VMEM((32768,), jnp.int32),
            pltpu.VMEM((CAP,), jnp.int32),
            pltpu.VMEM((16, 256), jnp.int32),
            pltpu.VMEM((256,), jnp.int32),
            pltpu.VMEM((16, 16), jnp.int32),
            pltpu.VMEM((16,), jnp.int32),
            pltpu.VMEM((16,), jnp.float32),
        ],
    )
    return kern(*xs)[0]


def _finalize(stats):
    def body(s_ref, o_ref):
        st = s_ref[...]
        counts = st[:, 0:1]
        medians = st[:, 1:2]
        scale = TOTAL_AMOUNT / jnp.sum(medians)
        col0 = counts * scale
        elig = ((medians >= MIN_AMOUNT) & (col0 >= QUORUM)).astype(jnp.float32)
        o_ref[...] = jnp.concatenate([col0, medians, elig], axis=1)
    return pl.pallas_call(
        body, out_shape=jax.ShapeDtypeStruct((NP, 3), jnp.float32))(stats)


@jax.jit
def kernel(x0, x1, x2, x3, x4, x5, x6, x7):
    flat = [v.reshape(N * 2) for v in (x0, x1, x2, x3, x4, x5, x6, x7)]
    return _finalize(_sc_stats(*flat))
